# baseline (speedup 1.0000x reference)
import functools

import jax
import jax.numpy as jnp
from jax import lax
from jax.experimental import pallas as pl
from jax.experimental.pallas import tpu as pltpu
from jax.experimental.pallas import tpu_sc as plsc

_NC = 2
_NS = 16
_NW = _NC * _NS

_B = 16384
_PROD_DIM = 32
_CAT_DIM = 16
_IN_DIM = 48
_OUT_DIM = 32
_BPW = _B // _NW

_NPROD = 100001
_RPAD = 100352
_QUARTER = _RPAD // 4
_RBLK = 12544

_NCAT = 1001
_CPAD = 1024
_CQUARTER = _CPAD // 4


def _pw_body(t0_ref, t1_ref, t2_ref, t3_ref, wt_ref, b_ref, o_ref):
    w1t = wt_ref[:, 0:_PROD_DIM]
    dn = (((0,), (1,)), ((), ()))
    for p, t_ref in enumerate((t0_ref, t1_ref, t2_ref, t3_ref)):
        o_ref[:, 32 * p:32 * (p + 1)] = lax.dot_general(
            t_ref[...], w1t, dn,
            preferred_element_type=jnp.float32) + b_ref[...]


def _pw_repack(tab_t, w_t, b2):
    grid = _QUARTER // _RBLK

    def spec(p):
        return pl.BlockSpec((_PROD_DIM, _RBLK),
                            lambda i, p=p: (0, p * grid + i))

    return pl.pallas_call(
        _pw_body,
        grid=(grid,),
        in_specs=[
            spec(0), spec(1), spec(2), spec(3),
            pl.BlockSpec((_OUT_DIM, _IN_DIM), lambda i: (0, 0)),
            pl.BlockSpec((1, _OUT_DIM), lambda i: (0, 0)),
        ],
        out_specs=pl.BlockSpec((_RBLK, 4 * _PROD_DIM), lambda i: (i, 0)),
        out_shape=jax.ShapeDtypeStruct((_QUARTER, 4 * _PROD_DIM),
                                       jnp.float32),
    )(tab_t, tab_t, tab_t, tab_t, w_t, b2)


def _cw_body(t0_ref, t1_ref, t2_ref, t3_ref, wt_ref, o_ref):
    w2t = wt_ref[:, _PROD_DIM:_IN_DIM]
    dn = (((0,), (1,)), ((), ()))
    for p, t_ref in enumerate((t0_ref, t1_ref, t2_ref, t3_ref)):
        o_ref[:, 32 * p:32 * (p + 1)] = lax.dot_general(
            t_ref[...], w2t, dn, preferred_element_type=jnp.float32)


def _cw_repack(ctab_t, w_t):
    def spec(p):
        return pl.BlockSpec((_CAT_DIM, _CQUARTER),
                            lambda i, p=p: (0, p))

    return pl.pallas_call(
        _cw_body,
        grid=(1,),
        in_specs=[
            spec(0), spec(1), spec(2), spec(3),
            pl.BlockSpec((_OUT_DIM, _IN_DIM), lambda i: (0, 0)),
        ],
        out_specs=pl.BlockSpec((_CQUARTER, 4 * _OUT_DIM), lambda i: (i, 0)),
        out_shape=jax.ShapeDtypeStruct((_CQUARTER, 4 * _OUT_DIM),
                                       jnp.float32),
    )(ctab_t, ctab_t, ctab_t, ctab_t, w_t)


def _sc_gather_body(pid_hbm, cat_hbm, ptab_hbm, ctab_hbm, out_hbm,
                    pidx_v, cidx_v, prow_v, crow_v, sem_p, sem_c):
    wid = lax.axis_index("s") * _NC + lax.axis_index("c")
    base = wid * _BPW
    pltpu.sync_copy(pid_hbm.at[pl.ds(base, _BPW)], pidx_v)
    pltpu.sync_copy(cat_hbm.at[pl.ds(base, _BPW)], cidx_v)
    cp_p = pltpu.async_copy(ptab_hbm.at[pidx_v], prow_v, sem_p)
    cp_c = pltpu.async_copy(ctab_hbm.at[cidx_v], crow_v, sem_c)
    cp_p.wait()
    cp_c.wait()

    def addrow(i, carry):
        pr = prow_v.at[i]
        cr = crow_v.at[i]
        pr[pl.ds(0, 16)] = pr[pl.ds(0, 16)] + cr[pl.ds(0, 16)]
        pr[pl.ds(16, 16)] = pr[pl.ds(16, 16)] + cr[pl.ds(16, 16)]
        return carry

    lax.fori_loop(0, _BPW, addrow, 0)
    pltpu.sync_copy(prow_v, out_hbm.at[pl.ds(base, _BPW)])


_sc_gather = pl.kernel(
    _sc_gather_body,
    out_type=jax.ShapeDtypeStruct((_B, _OUT_DIM), jnp.float32),
    name="sc_gather_add",
    mesh=plsc.VectorSubcoreMesh(core_axis_name="c", subcore_axis_name="s"),
    scratch_types=[
        pltpu.VMEM((_BPW,), jnp.int32),
        pltpu.VMEM((_BPW,), jnp.int32),
        pltpu.VMEM((_BPW, _OUT_DIM), jnp.float32),
        pltpu.VMEM((_BPW, _OUT_DIM), jnp.float32),
        pltpu.SemaphoreType.DMA,
        pltpu.SemaphoreType.DMA,
    ],
    compiler_params=pltpu.CompilerParams(use_tc_tiling_on_sc=False),
)


@jax.jit
def kernel(product_id, category, product_table, category_table, W, b):
    w_t = W.T
    b2 = b.reshape(1, _OUT_DIM)
    pw = _pw_repack(product_table.T, w_t, b2).reshape(_RPAD, _OUT_DIM)
    cw = _cw_repack(category_table.T, w_t).reshape(_CPAD, _OUT_DIM)
    pid2 = (product_id % _QUARTER) * 4 + product_id // _QUARTER
    cid2 = (category % _CQUARTER) * 4 + category // _CQUARTER
    return _sc_gather(pid2, cid2, pw, cw)


# speedup vs baseline: 1.0145x; 1.0145x over previous
import functools

import jax
import jax.numpy as jnp
from jax import lax
from jax.experimental import pallas as pl
from jax.experimental.pallas import tpu as pltpu
from jax.experimental.pallas import tpu_sc as plsc

_NC = 2
_NS = 16
_NW = _NC * _NS

_B = 16384
_PROD_DIM = 32
_CAT_DIM = 16
_IN_DIM = 48
_OUT_DIM = 32
_BPW = _B // _NW

_NPROD = 100001
_RPAD = 100352
_QUARTER = _RPAD // 4
_RBLK = 1792

_NCAT = 1001
_CPAD = 1024
_CQUARTER = _CPAD // 4


def _pw_body(t0_ref, t1_ref, t2_ref, t3_ref, wt_ref, b_ref, o_ref):
    w1t = wt_ref[:, 0:_PROD_DIM]
    dn = (((0,), (1,)), ((), ()))
    for p, t_ref in enumerate((t0_ref, t1_ref, t2_ref, t3_ref)):
        o_ref[:, 32 * p:32 * (p + 1)] = lax.dot_general(
            t_ref[...], w1t, dn,
            preferred_element_type=jnp.float32) + b_ref[...]


def _pw_repack(tab_t, w_t, b2):
    grid = _QUARTER // _RBLK

    def spec(p):
        return pl.BlockSpec((_PROD_DIM, _RBLK),
                            lambda i, p=p: (0, p * grid + i))

    return pl.pallas_call(
        _pw_body,
        grid=(grid,),
        in_specs=[
            spec(0), spec(1), spec(2), spec(3),
            pl.BlockSpec((_OUT_DIM, _IN_DIM), lambda i: (0, 0)),
            pl.BlockSpec((1, _OUT_DIM), lambda i: (0, 0)),
        ],
        out_specs=pl.BlockSpec((_RBLK, 4 * _PROD_DIM), lambda i: (i, 0)),
        out_shape=jax.ShapeDtypeStruct((_QUARTER, 4 * _PROD_DIM),
                                       jnp.float32),
    )(tab_t, tab_t, tab_t, tab_t, w_t, b2)


def _cw_body(t0_ref, t1_ref, t2_ref, t3_ref, wt_ref, o_ref):
    w2t = wt_ref[:, _PROD_DIM:_IN_DIM]
    dn = (((0,), (1,)), ((), ()))
    for p, t_ref in enumerate((t0_ref, t1_ref, t2_ref, t3_ref)):
        o_ref[:, 32 * p:32 * (p + 1)] = lax.dot_general(
            t_ref[...], w2t, dn, preferred_element_type=jnp.float32)


def _cw_repack(ctab_t, w_t):
    def spec(p):
        return pl.BlockSpec((_CAT_DIM, _CQUARTER),
                            lambda i, p=p: (0, p))

    return pl.pallas_call(
        _cw_body,
        grid=(1,),
        in_specs=[
            spec(0), spec(1), spec(2), spec(3),
            pl.BlockSpec((_OUT_DIM, _IN_DIM), lambda i: (0, 0)),
        ],
        out_specs=pl.BlockSpec((_CQUARTER, 4 * _OUT_DIM), lambda i: (i, 0)),
        out_shape=jax.ShapeDtypeStruct((_CQUARTER, 4 * _OUT_DIM),
                                       jnp.float32),
    )(ctab_t, ctab_t, ctab_t, ctab_t, w_t)


def _sc_gather_body(pid_hbm, cat_hbm, ptab_hbm, ctab_hbm, out_hbm,
                    pidx_v, cidx_v, prow_v, crow_v, sem_p, sem_c):
    wid = lax.axis_index("s") * _NC + lax.axis_index("c")
    base = wid * _BPW
    pltpu.sync_copy(pid_hbm.at[pl.ds(base, _BPW)], pidx_v)
    pltpu.sync_copy(cat_hbm.at[pl.ds(base, _BPW)], cidx_v)
    cp_p = pltpu.async_copy(ptab_hbm.at[pidx_v], prow_v, sem_p)
    cp_c = pltpu.async_copy(ctab_hbm.at[cidx_v], crow_v, sem_c)
    cp_p.wait()
    cp_c.wait()

    def addrow(i, carry):
        pr = prow_v.at[i]
        cr = crow_v.at[i]
        pr[pl.ds(0, 16)] = pr[pl.ds(0, 16)] + cr[pl.ds(0, 16)]
        pr[pl.ds(16, 16)] = pr[pl.ds(16, 16)] + cr[pl.ds(16, 16)]
        return carry

    lax.fori_loop(0, _BPW, addrow, 0)
    pltpu.sync_copy(prow_v, out_hbm.at[pl.ds(base, _BPW)])


_sc_gather = pl.kernel(
    _sc_gather_body,
    out_type=jax.ShapeDtypeStruct((_B, _OUT_DIM), jnp.float32),
    name="sc_gather_add",
    mesh=plsc.VectorSubcoreMesh(core_axis_name="c", subcore_axis_name="s"),
    scratch_types=[
        pltpu.VMEM((_BPW,), jnp.int32),
        pltpu.VMEM((_BPW,), jnp.int32),
        pltpu.VMEM((_BPW, _OUT_DIM), jnp.float32),
        pltpu.VMEM((_BPW, _OUT_DIM), jnp.float32),
        pltpu.SemaphoreType.DMA,
        pltpu.SemaphoreType.DMA,
    ],
    compiler_params=pltpu.CompilerParams(use_tc_tiling_on_sc=False),
)


@jax.jit
def kernel(product_id, category, product_table, category_table, W, b):
    w_t = W.T
    b2 = b.reshape(1, _OUT_DIM)
    pw = _pw_repack(product_table.T, w_t, b2).reshape(_RPAD, _OUT_DIM)
    cw = _cw_repack(category_table.T, w_t).reshape(_CPAD, _OUT_DIM)
    pid2 = (product_id % _QUARTER) * 4 + product_id // _QUARTER
    cid2 = (category % _CQUARTER) * 4 + category // _CQUARTER
    return _sc_gather(pid2, cid2, pw, cw)


# speedup vs baseline: 1.0302x; 1.0154x over previous
import functools

import jax
import jax.numpy as jnp
from jax import lax
from jax.experimental import pallas as pl
from jax.experimental.pallas import tpu as pltpu
from jax.experimental.pallas import tpu_sc as plsc

_NC = 2
_NS = 16
_NW = _NC * _NS

_B = 16384
_PROD_DIM = 32
_CAT_DIM = 16
_IN_DIM = 48
_OUT_DIM = 32
_BPW = _B // _NW

_NPROD = 100001
_RPAD = 100352
_QUARTER = _RPAD // 4
_RBLK = 3584

_NCAT = 1001
_CPAD = 1024
_CQUARTER = _CPAD // 4


def _pw_body(t_ref, wt_ref, b_ref, o_ref):
    w1t = wt_ref[:, 0:_PROD_DIM]
    dn = (((0,), (1,)), ((), ()))
    for p in range(4):
        o_ref[:, 32 * p:32 * (p + 1)] = lax.dot_general(
            t_ref[:, p * _RBLK:(p + 1) * _RBLK], w1t, dn,
            preferred_element_type=jnp.float32) + b_ref[...]


def _pw_repack(tab_t, w_t, b2):
    grid = _QUARTER // _RBLK
    return pl.pallas_call(
        _pw_body,
        grid=(grid,),
        in_specs=[
            pl.BlockSpec((_PROD_DIM, 4 * _RBLK), lambda i: (0, i)),
            pl.BlockSpec((_OUT_DIM, _IN_DIM), lambda i: (0, 0)),
            pl.BlockSpec((1, _OUT_DIM), lambda i: (0, 0)),
        ],
        out_specs=pl.BlockSpec((_RBLK, 4 * _PROD_DIM), lambda i: (i, 0)),
        out_shape=jax.ShapeDtypeStruct((_QUARTER, 4 * _PROD_DIM),
                                       jnp.float32),
    )(tab_t, w_t, b2)


def _cw_body(t0_ref, t1_ref, t2_ref, t3_ref, wt_ref, o_ref):
    w2t = wt_ref[:, _PROD_DIM:_IN_DIM]
    dn = (((0,), (1,)), ((), ()))
    for p, t_ref in enumerate((t0_ref, t1_ref, t2_ref, t3_ref)):
        o_ref[:, 32 * p:32 * (p + 1)] = lax.dot_general(
            t_ref[...], w2t, dn, preferred_element_type=jnp.float32)


def _cw_repack(ctab_t, w_t):
    def spec(p):
        return pl.BlockSpec((_CAT_DIM, _CQUARTER),
                            lambda i, p=p: (0, p))

    return pl.pallas_call(
        _cw_body,
        grid=(1,),
        in_specs=[
            spec(0), spec(1), spec(2), spec(3),
            pl.BlockSpec((_OUT_DIM, _IN_DIM), lambda i: (0, 0)),
        ],
        out_specs=pl.BlockSpec((_CQUARTER, 4 * _OUT_DIM), lambda i: (i, 0)),
        out_shape=jax.ShapeDtypeStruct((_CQUARTER, 4 * _OUT_DIM),
                                       jnp.float32),
    )(ctab_t, ctab_t, ctab_t, ctab_t, w_t)


def _sc_gather_body(pid_hbm, cat_hbm, ptab_hbm, ctab_hbm, out_hbm,
                    pidx_v, cidx_v, prow_v, crow_v, sem_p, sem_c):
    wid = lax.axis_index("s") * _NC + lax.axis_index("c")
    base = wid * _BPW
    pltpu.sync_copy(pid_hbm.at[pl.ds(base, _BPW)], pidx_v)
    pltpu.sync_copy(cat_hbm.at[pl.ds(base, _BPW)], cidx_v)
    cp_p = pltpu.async_copy(ptab_hbm.at[pidx_v], prow_v, sem_p)
    cp_c = pltpu.async_copy(ctab_hbm.at[cidx_v], crow_v, sem_c)
    cp_p.wait()
    cp_c.wait()

    def addrow(i, carry):
        pr = prow_v.at[i]
        cr = crow_v.at[i]
        pr[pl.ds(0, 16)] = pr[pl.ds(0, 16)] + cr[pl.ds(0, 16)]
        pr[pl.ds(16, 16)] = pr[pl.ds(16, 16)] + cr[pl.ds(16, 16)]
        return carry

    lax.fori_loop(0, _BPW, addrow, 0)
    pltpu.sync_copy(prow_v, out_hbm.at[pl.ds(base, _BPW)])


_sc_gather = pl.kernel(
    _sc_gather_body,
    out_type=jax.ShapeDtypeStruct((_B, _OUT_DIM), jnp.float32),
    name="sc_gather_add",
    mesh=plsc.VectorSubcoreMesh(core_axis_name="c", subcore_axis_name="s"),
    scratch_types=[
        pltpu.VMEM((_BPW,), jnp.int32),
        pltpu.VMEM((_BPW,), jnp.int32),
        pltpu.VMEM((_BPW, _OUT_DIM), jnp.float32),
        pltpu.VMEM((_BPW, _OUT_DIM), jnp.float32),
        pltpu.SemaphoreType.DMA,
        pltpu.SemaphoreType.DMA,
    ],
    compiler_params=pltpu.CompilerParams(use_tc_tiling_on_sc=False),
)


@jax.jit
def kernel(product_id, category, product_table, category_table, W, b):
    w_t = W.T
    b2 = b.reshape(1, _OUT_DIM)
    pw = _pw_repack(product_table.T, w_t, b2).reshape(_RPAD, _OUT_DIM)
    cw = _cw_repack(category_table.T, w_t).reshape(_CPAD, _OUT_DIM)
    blk = product_id // (4 * _RBLK)
    m = product_id % (4 * _RBLK)
    pid2 = 4 * (blk * _RBLK + m % _RBLK) + m // _RBLK
    cid2 = (category % _CQUARTER) * 4 + category // _CQUARTER
    return _sc_gather(pid2, cid2, pw, cw)


# speedup vs baseline: 1.0329x; 1.0026x over previous
import jax
import jax.numpy as jnp
from jax import lax
from jax.experimental import pallas as pl
from jax.experimental.pallas import tpu as pltpu
from jax.experimental.pallas import tpu_sc as plsc

_NC = 2
_NS = 16
_NW = _NC * _NS

_B = 16384
_PROD_DIM = 32
_CAT_DIM = 16
_IN_DIM = 48
_OUT_DIM = 32
_BPW = _B // _NW

_NPROD = 100001
_RPAD = 100352
_QUARTER = _RPAD // 4
_RBLK = 3584

_NCAT = 1001
_CPAD = 1024
_CQUARTER = _CPAD // 4


def _pw_body(t_ref, wt_ref, b_ref, o_ref):
    w1t = wt_ref[:, 0:_PROD_DIM]
    dn = (((0,), (1,)), ((), ()))
    for p in range(4):
        o_ref[:, 32 * p:32 * (p + 1)] = lax.dot_general(
            t_ref[:, p * _RBLK:(p + 1) * _RBLK], w1t, dn,
            preferred_element_type=jnp.float32) + b_ref[...]


def _pw_repack(tab_t, w_t, b2):
    grid = _QUARTER // _RBLK
    return pl.pallas_call(
        _pw_body,
        grid=(grid,),
        in_specs=[
            pl.BlockSpec((_PROD_DIM, 4 * _RBLK), lambda i: (0, i)),
            pl.BlockSpec((_OUT_DIM, _IN_DIM), lambda i: (0, 0)),
            pl.BlockSpec((1, _OUT_DIM), lambda i: (0, 0)),
        ],
        out_specs=pl.BlockSpec((_RBLK, 4 * _PROD_DIM), lambda i: (i, 0)),
        out_shape=jax.ShapeDtypeStruct((_QUARTER, 4 * _PROD_DIM),
                                       jnp.float32),
    )(tab_t, w_t, b2)


def _cw_body(t0_ref, t1_ref, t2_ref, t3_ref, wt_ref, o_ref):
    w2t = wt_ref[:, _PROD_DIM:_IN_DIM]
    dn = (((0,), (1,)), ((), ()))
    for p, t_ref in enumerate((t0_ref, t1_ref, t2_ref, t3_ref)):
        o_ref[:, 32 * p:32 * (p + 1)] = lax.dot_general(
            t_ref[...], w2t, dn, preferred_element_type=jnp.float32)


def _cw_repack(ctab_t, w_t):
    def spec(p):
        return pl.BlockSpec((_CAT_DIM, _CQUARTER),
                            lambda i, p=p: (0, p))

    return pl.pallas_call(
        _cw_body,
        grid=(1,),
        in_specs=[
            spec(0), spec(1), spec(2), spec(3),
            pl.BlockSpec((_OUT_DIM, _IN_DIM), lambda i: (0, 0)),
        ],
        out_specs=pl.BlockSpec((_CQUARTER, 4 * _OUT_DIM), lambda i: (i, 0)),
        out_shape=jax.ShapeDtypeStruct((_CQUARTER, 4 * _OUT_DIM),
                                       jnp.float32),
    )(ctab_t, ctab_t, ctab_t, ctab_t, w_t)


def _sc_gather_body(pid_hbm, cat_hbm, ptab_hbm, ctab_hbm, out_hbm,
                    pidx_v, cidx_v, prow_v, crow_v, sem_p, sem_c):
    wid = lax.axis_index("s") * _NC + lax.axis_index("c")
    base = wid * _BPW
    pltpu.sync_copy(pid_hbm.at[pl.ds(base, _BPW)], pidx_v)
    pltpu.sync_copy(cat_hbm.at[pl.ds(base, _BPW)], cidx_v)
    cp_p = pltpu.async_copy(ptab_hbm.at[pidx_v], prow_v, sem_p)
    cp_c = pltpu.async_copy(ctab_hbm.at[cidx_v], crow_v, sem_c)
    cp_p.wait()
    cp_c.wait()

    def addrow(i, carry):
        pr = prow_v.at[i]
        cr = crow_v.at[i]
        pr[pl.ds(0, 16)] = pr[pl.ds(0, 16)] + cr[pl.ds(0, 16)]
        pr[pl.ds(16, 16)] = pr[pl.ds(16, 16)] + cr[pl.ds(16, 16)]
        return carry

    lax.fori_loop(0, _BPW, addrow, 0)
    pltpu.sync_copy(prow_v, out_hbm.at[pl.ds(base, _BPW)])


_sc_gather = pl.kernel(
    _sc_gather_body,
    out_type=jax.ShapeDtypeStruct((_B, _OUT_DIM), jnp.float32),
    name="sc_gather_add",
    mesh=plsc.VectorSubcoreMesh(core_axis_name="c", subcore_axis_name="s"),
    scratch_types=[
        pltpu.VMEM((_BPW,), jnp.int32),
        pltpu.VMEM((_BPW,), jnp.int32),
        pltpu.VMEM((_BPW, _OUT_DIM), jnp.float32),
        pltpu.VMEM((_BPW, _OUT_DIM), jnp.float32),
        pltpu.SemaphoreType.DMA,
        pltpu.SemaphoreType.DMA,
    ],
    compiler_params=pltpu.CompilerParams(use_tc_tiling_on_sc=False),
)


@jax.jit
def kernel(product_id, category, product_table, category_table, W, b):
    w_t = W.T
    b2 = b.reshape(1, _OUT_DIM)
    pw = _pw_repack(product_table.T, w_t, b2).reshape(_RPAD, _OUT_DIM)
    cw = _cw_repack(category_table.T, w_t).reshape(_CPAD, _OUT_DIM)
    blk = product_id // (4 * _RBLK)
    m = product_id % (4 * _RBLK)
    pid2 = 4 * (blk * _RBLK + m % _RBLK) + m // _RBLK
    cid2 = (category % _CQUARTER) * 4 + category // _CQUARTER
    return _sc_gather(pid2, cid2, pw, cw)
